# Initial kernel scaffold; baseline (speedup 1.0000x reference)
import jax
import jax.numpy as jnp
from jax.experimental import pallas as pl


def kernel(input1, input2, table):
    raise NotImplementedError("write your pallas kernel here")


# speedup vs baseline: 1.2691x; 1.2691x over previous
import functools

import jax
import jax.numpy as jnp
from jax import lax
from jax.experimental import pallas as pl
from jax.experimental.pallas import tpu as pltpu
from jax.experimental.pallas import tpu_sc as plsc

VOCAB = 1000000
D = 64
B = 16384
L_SEQ = 20
EPS = 1e-6

NC = 2
NS = 16
LANES = 16
NW = NC * NS
B_PER_W = B // NW
CB = 16
NCH = B_PER_W // CB
NG = D // LANES


def _rsqrt_newton(x):
    i = plsc.bitcast(x, jnp.int32)
    i = jnp.int32(0x5F3759DF) - lax.shift_right_arithmetic(i, jnp.int32(1))
    y = plsc.bitcast(i, jnp.float32)
    for _ in range(3):
        y = y * (1.5 - 0.5 * (x * y) * y)
    return y


@functools.cache
def _build_sc_cosine():
    mesh = plsc.VectorSubcoreMesh(core_axis_name="c", subcore_axis_name="s")

    @functools.partial(
        pl.kernel,
        mesh=mesh,
        out_type=jax.ShapeDtypeStruct((B,), jnp.float32),
        compiler_params=pltpu.CompilerParams(
            needs_layout_passes=False, use_tc_tiling_on_sc=False),
        scratch_types=[
            pltpu.VMEM((CB * L_SEQ,), jnp.int32),
            pltpu.VMEM((CB * L_SEQ,), jnp.int32),
            pltpu.VMEM((CB * L_SEQ, D), jnp.float32),
            pltpu.VMEM((CB * L_SEQ, D), jnp.float32),
            pltpu.VMEM((B_PER_W,), jnp.float32),
            pltpu.SemaphoreType.DMA,
            pltpu.SemaphoreType.DMA,
        ],
    )
    def _sc_cosine(i1_hbm, i2_hbm, table_hbm, out_hbm,
                   idx1_v, idx2_v, rows1_v, rows2_v,
                   out_v, sem1, sem2):
        wid = lax.axis_index("s") * NC + lax.axis_index("c")
        base = wid * B_PER_W

        def chunk_body(c, carry):
            cbase = (base + c * CB) * L_SEQ
            pltpu.sync_copy(i1_hbm.at[pl.ds(cbase, CB * L_SEQ)], idx1_v)
            pltpu.sync_copy(i2_hbm.at[pl.ds(cbase, CB * L_SEQ)], idx2_v)
            cp1 = pltpu.async_copy(table_hbm.at[idx1_v], rows1_v, sem1)
            cp2 = pltpu.async_copy(table_hbm.at[idx2_v], rows2_v, sem2)
            cp1.wait()
            cp2.wait()

            lane = lax.iota(jnp.int32, LANES)

            def row_body(r, carry2):
                dot_t, s1_t, s2_t = carry2
                rb = r * L_SEQ
                a1 = [rows1_v[rb, pl.ds(g * LANES, LANES)] for g in range(NG)]
                a2 = [rows2_v[rb, pl.ds(g * LANES, LANES)] for g in range(NG)]
                for j in range(1, L_SEQ):
                    for g in range(NG):
                        a1[g] = a1[g] + rows1_v[rb + j, pl.ds(g * LANES, LANES)]
                        a2[g] = a2[g] + rows2_v[rb + j, pl.ds(g * LANES, LANES)]
                dotv = a1[0] * a2[0]
                s1v = a1[0] * a1[0]
                s2v = a2[0] * a2[0]
                for g in range(1, NG):
                    dotv = dotv + a1[g] * a2[g]
                    s1v = s1v + a1[g] * a1[g]
                    s2v = s2v + a2[g] * a2[g]
                m = lane == r
                dot_t = jnp.where(m, jnp.sum(dotv), dot_t)
                s1_t = jnp.where(m, jnp.sum(s1v), s1_t)
                s2_t = jnp.where(m, jnp.sum(s2v), s2_t)
                return dot_t, s1_t, s2_t

            zeros = jnp.zeros((LANES,), jnp.float32)
            dot_t, s1_t, s2_t = lax.fori_loop(
                0, CB, row_body, (zeros, zeros, zeros))

            inv_l = 1.0 / float(L_SEQ)
            dot_m = dot_t * (inv_l * inv_l)
            s_m = (s1_t * s2_t) * (inv_l * inv_l * inv_l * inv_l)
            rs = _rsqrt_newton(s_m)
            sqrt_m = s_m * rs
            denom = jnp.maximum(sqrt_m, EPS)
            out_v[pl.ds(c * CB, CB)] = dot_m / denom
            return carry

        lax.fori_loop(0, NCH, chunk_body, 0)
        pltpu.sync_copy(out_v, out_hbm.at[pl.ds(base, B_PER_W)])

    return _sc_cosine


def kernel(input1, input2, table):
    i1 = input1.reshape(-1)
    i2 = input2.reshape(-1)
    return _build_sc_cosine()(i1, i2, table)


# speedup vs baseline: 1.8596x; 1.4653x over previous
import functools

import jax
import jax.numpy as jnp
from jax import lax
from jax.experimental import pallas as pl
from jax.experimental.pallas import tpu as pltpu
from jax.experimental.pallas import tpu_sc as plsc

VOCAB = 1000000
D = 64
B = 16384
L_SEQ = 20
EPS = 1e-6

NC = 2
NS = 16
LANES = 16
NW = NC * NS
B_PER_W = B // NW
CB = 16
NCH = B_PER_W // CB
NG = D // LANES
NI = CB * L_SEQ
VB = 2048
NBLK = VOCAB // VB
C_TAIL = NBLK * VB
P_ROWS = C_TAIL // 2 + (VOCAB - C_TAIL)


def _rsqrt_newton(x):
    i = plsc.bitcast(x, jnp.int32)
    i = jnp.int32(0x5F3759DF) - lax.shift_right_arithmetic(i, jnp.int32(1))
    y = plsc.bitcast(i, jnp.float32)
    for _ in range(3):
        y = y * (1.5 - 0.5 * (x * y) * y)
    return y


@functools.cache
def _build_sc_cosine():
    mesh = plsc.VectorSubcoreMesh(core_axis_name="c", subcore_axis_name="s")

    @functools.partial(
        pl.kernel,
        mesh=mesh,
        out_type=jax.ShapeDtypeStruct((B,), jnp.float32),
        compiler_params=pltpu.CompilerParams(
            needs_layout_passes=False, use_tc_tiling_on_sc=False),
        scratch_types=[
            pltpu.VMEM((NI,), jnp.int32),
            pltpu.VMEM((NI,), jnp.int32),
            pltpu.VMEM((NI,), jnp.int32),
            pltpu.VMEM((NI,), jnp.int32),
            pltpu.VMEM((NI, D), jnp.float32),
            pltpu.VMEM((NI, D), jnp.float32),
            pltpu.VMEM((B_PER_W,), jnp.float32),
            pltpu.SemaphoreType.DMA,
            pltpu.SemaphoreType.DMA,
        ],
    )
    def _sc_cosine(i1_hbm, i2_hbm, table_hbm, out_hbm,
                   idx1_v, idx2_v, idxp1_v, idxp2_v,
                   rows1_v, rows2_v, out_v, sem1, sem2):
        wid = lax.axis_index("s") * NC + lax.axis_index("c")
        base = wid * B_PER_W

        def chunk_body(c, carry):
            cbase = (base + c * CB) * L_SEQ
            pltpu.sync_copy(i1_hbm.at[pl.ds(cbase, NI)], idx1_v)
            pltpu.sync_copy(i2_hbm.at[pl.ds(cbase, NI)], idx2_v)
            for v in range(NI // LANES):
                sl = pl.ds(v * LANES, LANES)
                for src, dst in ((idx1_v, idxp1_v), (idx2_v, idxp2_v)):
                    iv = src[sl]
                    r = iv & (2 * VB - 1)
                    main_lin = iv + r - jnp.where(
                        r < VB, 0, 2 * VB - 1).astype(jnp.int32)
                    tail_lin = iv * 2 - C_TAIL
                    dst[sl] = jnp.where(iv < C_TAIL, main_lin, tail_lin)
            cp1 = pltpu.async_copy(table_hbm.at[idxp1_v], rows1_v, sem1)
            cp2 = pltpu.async_copy(table_hbm.at[idxp2_v], rows2_v, sem2)
            cp1.wait()
            cp2.wait()

            lane = lax.iota(jnp.int32, LANES)

            def row_body(r, carry2):
                dot_t, s1_t, s2_t = carry2
                rb = r * L_SEQ
                a1 = [rows1_v[rb, pl.ds(g * LANES, LANES)] for g in range(NG)]
                a2 = [rows2_v[rb, pl.ds(g * LANES, LANES)] for g in range(NG)]
                for j in range(1, L_SEQ):
                    for g in range(NG):
                        a1[g] = a1[g] + rows1_v[
                            rb + j, pl.ds(g * LANES, LANES)]
                        a2[g] = a2[g] + rows2_v[
                            rb + j, pl.ds(g * LANES, LANES)]
                dotv = a1[0] * a2[0]
                s1v = a1[0] * a1[0]
                s2v = a2[0] * a2[0]
                for g in range(1, NG):
                    dotv = dotv + a1[g] * a2[g]
                    s1v = s1v + a1[g] * a1[g]
                    s2v = s2v + a2[g] * a2[g]
                m = lane == r
                dot_t = jnp.where(m, jnp.sum(dotv), dot_t)
                s1_t = jnp.where(m, jnp.sum(s1v), s1_t)
                s2_t = jnp.where(m, jnp.sum(s2v), s2_t)
                return dot_t, s1_t, s2_t

            zeros = jnp.zeros((LANES,), jnp.float32)
            dot_t, s1_t, s2_t = lax.fori_loop(
                0, CB, row_body, (zeros, zeros, zeros))

            inv_l = 1.0 / float(L_SEQ)
            dot_m = dot_t * (inv_l * inv_l)
            s_m = (s1_t * s2_t) * (inv_l * inv_l * inv_l * inv_l)
            rs = _rsqrt_newton(s_m)
            sqrt_m = s_m * rs
            denom = jnp.maximum(sqrt_m, EPS)
            out_v[pl.ds(c * CB, CB)] = dot_m / denom
            return carry

        lax.fori_loop(0, NCH, chunk_body, 0)
        pltpu.sync_copy(out_v, out_hbm.at[pl.ds(base, B_PER_W)])

    return _sc_cosine


@functools.cache
def _build_tc_relayout():
    grid = (P_ROWS + VB - 1) // VB

    def body(x1_ref, x2_ref, o_ref):
        lo = x1_ref[...].T
        hi = x2_ref[...].T
        o_ref[...] = jnp.concatenate([lo, hi], axis=1)

    return pl.pallas_call(
        body,
        grid=(grid,),
        in_specs=[
            pl.BlockSpec((D, VB), lambda i: (0, 2 * i)),
            pl.BlockSpec((D, VB), lambda i: (0, jnp.minimum(2 * i + 1, NBLK))),
        ],
        out_specs=pl.BlockSpec((VB, 2 * D), lambda i: (i, 0)),
        out_shape=jax.ShapeDtypeStruct((P_ROWS, 2 * D), jnp.float32),
    )


def kernel(input1, input2, table):
    i1 = input1.reshape(-1)
    i2 = input2.reshape(-1)
    tabt = table.T
    packed = _build_tc_relayout()(tabt, tabt)
    tab2 = packed.reshape(2 * P_ROWS, D)
    return _build_sc_cosine()(i1, i2, tab2)


# speedup vs baseline: 2.4159x; 1.2991x over previous
import functools

import jax
import jax.numpy as jnp
from jax import lax
from jax.experimental import pallas as pl
from jax.experimental.pallas import tpu as pltpu
from jax.experimental.pallas import tpu_sc as plsc

VOCAB = 1000000
D = 64
B = 16384
L_SEQ = 20
EPS = 1e-6

NC = 2
NS = 16
LANES = 16
NW = NC * NS
B_PER_W = B // NW
CB = 16
NCH = B_PER_W // CB
NI = CB * L_SEQ
W = D // 2
WG = W // LANES

VB = 2048
NBLK = VOCAB // VB
C_TAIL = NBLK * VB
P_ROWS = NBLK // 4 * VB + (VOCAB - C_TAIL)
N32 = 4 * P_ROWS


def _rsqrt_newton(x):
    i = plsc.bitcast(x, jnp.int32)
    i = jnp.int32(0x5F3759DF) - lax.shift_right_arithmetic(i, jnp.int32(1))
    y = plsc.bitcast(i, jnp.float32)
    for _ in range(3):
        y = y * (1.5 - 0.5 * (x * y) * y)
    return y


@functools.cache
def _build_sc_cosine():
    mesh = plsc.VectorSubcoreMesh(core_axis_name="c", subcore_axis_name="s")

    @functools.partial(
        pl.kernel,
        mesh=mesh,
        out_type=jax.ShapeDtypeStruct((B,), jnp.float32),
        compiler_params=pltpu.CompilerParams(
            needs_layout_passes=False, use_tc_tiling_on_sc=False),
        scratch_types=[
            pltpu.VMEM((NI,), jnp.int32),
            pltpu.VMEM((NI,), jnp.int32),
            pltpu.VMEM((NI,), jnp.int32),
            pltpu.VMEM((NI,), jnp.int32),
            pltpu.VMEM((NI, W), jnp.int32),
            pltpu.VMEM((NI, W), jnp.int32),
            pltpu.VMEM((B_PER_W,), jnp.float32),
            pltpu.SemaphoreType.DMA,
            pltpu.SemaphoreType.DMA,
        ],
    )
    def _sc_cosine(i1_hbm, i2_hbm, table_hbm, out_hbm,
                   idx1_v, idx2_v, idxp1_v, idxp2_v,
                   rows1_v, rows2_v, out_v, sem1, sem2):
        wid = lax.axis_index("s") * NC + lax.axis_index("c")
        base = wid * B_PER_W
        hi_mask = jnp.int32(-65536)

        def unpack(wv):
            hi = plsc.bitcast(wv & hi_mask, jnp.float32)
            lo = plsc.bitcast(lax.shift_left(wv, 16), jnp.float32)
            return hi, lo

        def chunk_body(c, carry):
            cbase = (base + c * CB) * L_SEQ
            pltpu.sync_copy(i1_hbm.at[pl.ds(cbase, NI)], idx1_v)
            pltpu.sync_copy(i2_hbm.at[pl.ds(cbase, NI)], idx2_v)
            for v in range(NI // LANES):
                sl = pl.ds(v * LANES, LANES)
                for src, dst in ((idx1_v, idxp1_v), (idx2_v, idxp2_v)):
                    iv = src[sl]
                    r = iv & (4 * VB - 1)
                    main_lin = (iv - r) + 4 * (r & (VB - 1)) \
                        + lax.shift_right_logical(r, 11)
                    tail_lin = iv * 4 - 3 * C_TAIL
                    dst[sl] = jnp.where(iv < C_TAIL, main_lin, tail_lin)
            cp1 = pltpu.async_copy(table_hbm.at[idxp1_v], rows1_v, sem1)
            cp2 = pltpu.async_copy(table_hbm.at[idxp2_v], rows2_v, sem2)
            cp1.wait()
            cp2.wait()

            lane = lax.iota(jnp.int32, LANES)

            def row_body(r, carry2):
                dot_t, s1_t, s2_t = carry2
                rb = r * L_SEQ
                a1 = []
                a2 = []
                for g in range(WG):
                    h, l = unpack(rows1_v[rb, pl.ds(g * LANES, LANES)])
                    a1 += [h, l]
                    h, l = unpack(rows2_v[rb, pl.ds(g * LANES, LANES)])
                    a2 += [h, l]
                for j in range(1, L_SEQ):
                    for g in range(WG):
                        h, l = unpack(
                            rows1_v[rb + j, pl.ds(g * LANES, LANES)])
                        a1[2 * g] = a1[2 * g] + h
                        a1[2 * g + 1] = a1[2 * g + 1] + l
                        h, l = unpack(
                            rows2_v[rb + j, pl.ds(g * LANES, LANES)])
                        a2[2 * g] = a2[2 * g] + h
                        a2[2 * g + 1] = a2[2 * g + 1] + l
                dotv = a1[0] * a2[0]
                s1v = a1[0] * a1[0]
                s2v = a2[0] * a2[0]
                for g in range(1, 2 * WG):
                    dotv = dotv + a1[g] * a2[g]
                    s1v = s1v + a1[g] * a1[g]
                    s2v = s2v + a2[g] * a2[g]
                m = lane == r
                dot_t = jnp.where(m, jnp.sum(dotv), dot_t)
                s1_t = jnp.where(m, jnp.sum(s1v), s1_t)
                s2_t = jnp.where(m, jnp.sum(s2v), s2_t)
                return dot_t, s1_t, s2_t

            zeros = jnp.zeros((LANES,), jnp.float32)
            dot_t, s1_t, s2_t = lax.fori_loop(
                0, CB, row_body, (zeros, zeros, zeros))

            inv_l = 1.0 / float(L_SEQ)
            dot_m = dot_t * (inv_l * inv_l)
            s_m = (s1_t * s2_t) * (inv_l * inv_l * inv_l * inv_l)
            rs = _rsqrt_newton(s_m)
            sqrt_m = s_m * rs
            denom = jnp.maximum(sqrt_m, EPS)
            out_v[pl.ds(c * CB, CB)] = dot_m / denom
            return carry

        lax.fori_loop(0, NCH, chunk_body, 0)
        pltpu.sync_copy(out_v, out_hbm.at[pl.ds(base, B_PER_W)])

    return _sc_cosine


@functools.cache
def _build_tc_relayout():
    grid = P_ROWS // VB + 1

    def pack16(x):
        t = x[:W, :].astype(jnp.bfloat16).astype(jnp.float32)
        b = x[W:, :].astype(jnp.bfloat16).astype(jnp.float32)
        ti = lax.bitcast_convert_type(t, jnp.int32)
        bi = lax.bitcast_convert_type(b, jnp.int32)
        return ti | lax.shift_right_logical(bi, 16)

    def body(x0_ref, x1_ref, x2_ref, x3_ref, o_ref):
        parts = [pack16(r[...]).T for r in (x0_ref, x1_ref, x2_ref, x3_ref)]
        o_ref[...] = jnp.concatenate(parts, axis=1)

    def spec(k):
        return pl.BlockSpec(
            (D, VB), lambda i, k=k: (0, jnp.minimum(4 * i + k, NBLK)))

    return pl.pallas_call(
        body,
        grid=(grid,),
        in_specs=[spec(0), spec(1), spec(2), spec(3)],
        out_specs=pl.BlockSpec((VB, 4 * W), lambda i: (i, 0)),
        out_shape=jax.ShapeDtypeStruct((P_ROWS, 4 * W), jnp.int32),
    )


def kernel(input1, input2, table):
    i1 = input1.reshape(-1)
    i2 = input2.reshape(-1)
    tabt = table.T
    packed = _build_tc_relayout()(tabt, tabt, tabt, tabt)
    tab2 = packed.reshape(N32, W)
    return _build_sc_cosine()(i1, i2, tab2)


# speedup vs baseline: 2.7512x; 1.1388x over previous
import functools

import jax
import jax.numpy as jnp
from jax import lax
from jax.experimental import pallas as pl
from jax.experimental.pallas import tpu as pltpu
from jax.experimental.pallas import tpu_sc as plsc

VOCAB = 1000000
D = 64
B = 16384
L_SEQ = 20
EPS = 1e-6

NC = 2
NS = 16
LANES = 16
NW = NC * NS
B_PER_W = B // NW
CB = 16
NCH = B_PER_W // CB
NI = CB * L_SEQ
W = D // 2
WG = W // LANES

VB = 4096
NBLK = VOCAB // VB
C_TAIL = NBLK * VB
P_ROWS = NBLK // 4 * VB + (VOCAB - C_TAIL)
N32 = 4 * P_ROWS


def _rsqrt_newton(x):
    i = plsc.bitcast(x, jnp.int32)
    i = jnp.int32(0x5F3759DF) - lax.shift_right_arithmetic(i, jnp.int32(1))
    y = plsc.bitcast(i, jnp.float32)
    for _ in range(3):
        y = y * (1.5 - 0.5 * (x * y) * y)
    return y


@functools.cache
def _build_sc_cosine():
    mesh = plsc.VectorSubcoreMesh(core_axis_name="c", subcore_axis_name="s")

    @functools.partial(
        pl.kernel,
        mesh=mesh,
        out_type=jax.ShapeDtypeStruct((B,), jnp.float32),
        compiler_params=pltpu.CompilerParams(
            needs_layout_passes=False, use_tc_tiling_on_sc=False),
        scratch_types=[
            pltpu.VMEM((NI,), jnp.int32),
            pltpu.VMEM((NI,), jnp.int32),
            pltpu.VMEM((NI,), jnp.int32),
            pltpu.VMEM((NI,), jnp.int32),
            pltpu.VMEM((NI, W), jnp.int32),
            pltpu.VMEM((NI, W), jnp.int32),
            pltpu.VMEM((NI,), jnp.int32),
            pltpu.VMEM((NI,), jnp.int32),
            pltpu.VMEM((NI,), jnp.int32),
            pltpu.VMEM((NI,), jnp.int32),
            pltpu.VMEM((NI, W), jnp.int32),
            pltpu.VMEM((NI, W), jnp.int32),
            pltpu.VMEM((B_PER_W,), jnp.float32),
            pltpu.SemaphoreType.DMA,
            pltpu.SemaphoreType.DMA,
            pltpu.SemaphoreType.DMA,
            pltpu.SemaphoreType.DMA,
        ],
    )
    def _sc_cosine(i1_hbm, i2_hbm, table_hbm, out_hbm,
                   idx1_a, idx2_a, idxp1_a, idxp2_a, rows1_a, rows2_a,
                   idx1_b, idx2_b, idxp1_b, idxp2_b, rows1_b, rows2_b,
                   out_v, sem1_a, sem2_a, sem1_b, sem2_b):
        wid = lax.axis_index("s") * NC + lax.axis_index("c")
        base = wid * B_PER_W
        hi_mask = jnp.int32(-65536)
        bufs = (
            (idx1_a, idx2_a, idxp1_a, idxp2_a, rows1_a, rows2_a,
             sem1_a, sem2_a),
            (idx1_b, idx2_b, idxp1_b, idxp2_b, rows1_b, rows2_b,
             sem1_b, sem2_b),
        )

        def unpack(wv):
            hi = plsc.bitcast(wv & hi_mask, jnp.float32)
            lo = plsc.bitcast(lax.shift_left(wv, 16), jnp.float32)
            return hi, lo

        def start_chunk(c, k):
            idx1_v, idx2_v, idxp1_v, idxp2_v, rows1_v, rows2_v, s1, s2 = \
                bufs[k]
            cbase = (base + c * CB) * L_SEQ
            pltpu.sync_copy(i1_hbm.at[pl.ds(cbase, NI)], idx1_v)
            pltpu.sync_copy(i2_hbm.at[pl.ds(cbase, NI)], idx2_v)
            for v in range(NI // LANES):
                sl = pl.ds(v * LANES, LANES)
                for src, dst in ((idx1_v, idxp1_v), (idx2_v, idxp2_v)):
                    iv = src[sl]
                    r = iv & (4 * VB - 1)
                    main_lin = (iv - r) + 4 * (r & (VB - 1)) \
                        + lax.shift_right_logical(r, VB.bit_length() - 1)
                    tail_lin = iv * 4 - 3 * C_TAIL
                    dst[sl] = jnp.where(iv < C_TAIL, main_lin, tail_lin)
            pltpu.async_copy(table_hbm.at[idxp1_v], rows1_v, s1)
            pltpu.async_copy(table_hbm.at[idxp2_v], rows2_v, s2)

        def process_chunk(c, k):
            _, _, idxp1_v, idxp2_v, rows1_v, rows2_v, s1, s2 = bufs[k]
            pltpu.make_async_copy(table_hbm.at[idxp1_v], rows1_v, s1).wait()
            pltpu.make_async_copy(table_hbm.at[idxp2_v], rows2_v, s2).wait()

            lane = lax.iota(jnp.int32, LANES)

            def row_body(r, carry2):
                dot_t, s1_t, s2_t = carry2
                rb = r * L_SEQ
                a1 = []
                a2 = []
                for g in range(WG):
                    h, l = unpack(rows1_v[rb, pl.ds(g * LANES, LANES)])
                    a1 += [h, l]
                    h, l = unpack(rows2_v[rb, pl.ds(g * LANES, LANES)])
                    a2 += [h, l]
                for j in range(1, L_SEQ):
                    for g in range(WG):
                        h, l = unpack(
                            rows1_v[rb + j, pl.ds(g * LANES, LANES)])
                        a1[2 * g] = a1[2 * g] + h
                        a1[2 * g + 1] = a1[2 * g + 1] + l
                        h, l = unpack(
                            rows2_v[rb + j, pl.ds(g * LANES, LANES)])
                        a2[2 * g] = a2[2 * g] + h
                        a2[2 * g + 1] = a2[2 * g + 1] + l
                dotv = a1[0] * a2[0]
                s1v = a1[0] * a1[0]
                s2v = a2[0] * a2[0]
                for g in range(1, 2 * WG):
                    dotv = dotv + a1[g] * a2[g]
                    s1v = s1v + a1[g] * a1[g]
                    s2v = s2v + a2[g] * a2[g]
                m = lane == r
                dot_t = jnp.where(m, jnp.sum(dotv), dot_t)
                s1_t = jnp.where(m, jnp.sum(s1v), s1_t)
                s2_t = jnp.where(m, jnp.sum(s2v), s2_t)
                return dot_t, s1_t, s2_t

            zeros = jnp.zeros((LANES,), jnp.float32)
            dot_t, s1_t, s2_t = lax.fori_loop(
                0, CB, row_body, (zeros, zeros, zeros))

            inv_l = 1.0 / float(L_SEQ)
            dot_m = dot_t * (inv_l * inv_l)
            s_m = (s1_t * s2_t) * (inv_l * inv_l * inv_l * inv_l)
            rs = _rsqrt_newton(s_m)
            sqrt_m = s_m * rs
            denom = jnp.maximum(sqrt_m, EPS)
            out_v[pl.ds(c * CB, CB)] = dot_m / denom

        start_chunk(0, 0)

        def pair_body(p, carry):
            c0 = 2 * p
            start_chunk(c0 + 1, 1)
            process_chunk(c0, 0)

            @pl.when(p + 1 < NCH // 2)
            def _():
                start_chunk(c0 + 2, 0)

            process_chunk(c0 + 1, 1)
            return carry

        lax.fori_loop(0, NCH // 2, pair_body, 0)
        pltpu.sync_copy(out_v, out_hbm.at[pl.ds(base, B_PER_W)])

    return _sc_cosine


@functools.cache
def _build_tc_relayout():
    grid = P_ROWS // VB + 1

    def pack16(x):
        t = x[:W, :].astype(jnp.bfloat16).astype(jnp.float32)
        b = x[W:, :].astype(jnp.bfloat16).astype(jnp.float32)
        ti = lax.bitcast_convert_type(t, jnp.int32)
        bi = lax.bitcast_convert_type(b, jnp.int32)
        return ti | lax.shift_right_logical(bi, 16)

    def body(x0_ref, x1_ref, x2_ref, x3_ref, o_ref):
        parts = [pack16(r[...]).T for r in (x0_ref, x1_ref, x2_ref, x3_ref)]
        o_ref[...] = jnp.concatenate(parts, axis=1)

    def spec(k):
        return pl.BlockSpec(
            (D, VB), lambda i, k=k: (0, jnp.minimum(4 * i + k, NBLK)))

    return pl.pallas_call(
        body,
        grid=(grid,),
        in_specs=[spec(0), spec(1), spec(2), spec(3)],
        out_specs=pl.BlockSpec((VB, 4 * W), lambda i: (i, 0)),
        out_shape=jax.ShapeDtypeStruct((P_ROWS, 4 * W), jnp.int32),
    )


def kernel(input1, input2, table):
    i1 = input1.reshape(-1)
    i2 = input2.reshape(-1)
    tabt = table.T
    packed = _build_tc_relayout()(tabt, tabt, tabt, tabt)
    tab2 = packed.reshape(N32, W)
    return _build_sc_cosine()(i1, i2, tab2)


# speedup vs baseline: 4.1583x; 1.5114x over previous
import functools

import jax
import jax.numpy as jnp
from jax import lax
from jax.experimental import pallas as pl
from jax.experimental.pallas import tpu as pltpu
from jax.experimental.pallas import tpu_sc as plsc

VOCAB = 1000000
D = 64
B = 16384
L_SEQ = 20
EPS = 1e-6

NC = 2
NS = 16
LANES = 16
NW = NC * NS
B_PER_W = B // NW
CB = 16
NCH = B_PER_W // CB
NI = CB * L_SEQ
W = D // 2
WG = W // LANES

VB = 4096
NBLK = VOCAB // VB
C_TAIL = NBLK * VB
P_ROWS = NBLK // 4 * VB + (VOCAB - C_TAIL)
N32 = 4 * P_ROWS


def _rsqrt_newton(x):
    i = plsc.bitcast(x, jnp.int32)
    i = jnp.int32(0x5F3759DF) - lax.shift_right_arithmetic(i, jnp.int32(1))
    y = plsc.bitcast(i, jnp.float32)
    for _ in range(3):
        y = y * (1.5 - 0.5 * (x * y) * y)
    return y


@functools.cache
def _build_sc_cosine():
    mesh = plsc.VectorSubcoreMesh(core_axis_name="c", subcore_axis_name="s")

    @functools.partial(
        pl.kernel,
        mesh=mesh,
        out_type=jax.ShapeDtypeStruct((B,), jnp.float32),
        compiler_params=pltpu.CompilerParams(
            needs_layout_passes=False, use_tc_tiling_on_sc=False),
        scratch_types=[
            pltpu.VMEM((NI,), jnp.int32),
            pltpu.VMEM((NI,), jnp.int32),
            pltpu.VMEM((NI,), jnp.int32),
            pltpu.VMEM((NI,), jnp.int32),
            pltpu.VMEM((NI, W), jnp.int32),
            pltpu.VMEM((NI, W), jnp.int32),
            pltpu.VMEM((NI,), jnp.int32),
            pltpu.VMEM((NI,), jnp.int32),
            pltpu.VMEM((NI,), jnp.int32),
            pltpu.VMEM((NI,), jnp.int32),
            pltpu.VMEM((NI, W), jnp.int32),
            pltpu.VMEM((NI, W), jnp.int32),
            pltpu.VMEM((B_PER_W,), jnp.float32),
            pltpu.SemaphoreType.DMA,
            pltpu.SemaphoreType.DMA,
            pltpu.SemaphoreType.DMA,
            pltpu.SemaphoreType.DMA,
        ],
    )
    def _sc_cosine(i1_hbm, i2_hbm, table_hbm, out_hbm,
                   idx1_a, idx2_a, idxp1_a, idxp2_a, rows1_a, rows2_a,
                   idx1_b, idx2_b, idxp1_b, idxp2_b, rows1_b, rows2_b,
                   out_v, sem1_a, sem2_a, sem1_b, sem2_b):
        wid = lax.axis_index("s") * NC + lax.axis_index("c")
        base = wid * B_PER_W
        hi_mask = jnp.int32(-65536)
        bufs = (
            (idx1_a, idx2_a, idxp1_a, idxp2_a, rows1_a, rows2_a,
             sem1_a, sem2_a),
            (idx1_b, idx2_b, idxp1_b, idxp2_b, rows1_b, rows2_b,
             sem1_b, sem2_b),
        )

        def unpack(wv):
            hi = plsc.bitcast(wv & hi_mask, jnp.float32)
            lo = plsc.bitcast(lax.shift_left(wv, 16), jnp.float32)
            return hi, lo

        def start_chunk(c, k):
            idx1_v, idx2_v, idxp1_v, idxp2_v, rows1_v, rows2_v, s1, s2 = \
                bufs[k]
            cbase = (base + c * CB) * L_SEQ
            pltpu.sync_copy(i1_hbm.at[pl.ds(cbase, NI)], idx1_v)
            pltpu.sync_copy(i2_hbm.at[pl.ds(cbase, NI)], idx2_v)
            for v in range(NI // LANES):
                sl = pl.ds(v * LANES, LANES)
                for src, dst in ((idx1_v, idxp1_v), (idx2_v, idxp2_v)):
                    iv = src[sl]
                    r = iv & (4 * VB - 1)
                    main_lin = (iv - r) + 4 * (r & (VB - 1)) \
                        + lax.shift_right_logical(r, VB.bit_length() - 1)
                    tail_lin = iv * 4 - 3 * C_TAIL
                    dst[sl] = jnp.where(iv < C_TAIL, main_lin, tail_lin)
            pltpu.async_copy(table_hbm.at[idxp1_v], rows1_v, s1)
            pltpu.async_copy(table_hbm.at[idxp2_v], rows2_v, s2)

        def process_chunk(c, k):
            _, _, idxp1_v, idxp2_v, rows1_v, rows2_v, s1, s2 = bufs[k]
            pltpu.make_async_copy(table_hbm.at[idxp1_v], rows1_v, s1).wait()
            pltpu.make_async_copy(table_hbm.at[idxp2_v], rows2_v, s2).wait()

            lane = lax.iota(jnp.int32, LANES)

            def row_body(r, carry2):
                dot_t, s1_t, s2_t = carry2
                rb = r * L_SEQ
                a1 = []
                a2 = []
                for g in range(WG):
                    h, l = unpack(rows1_v[rb, pl.ds(g * LANES, LANES)])
                    a1 += [h, l]
                    h, l = unpack(rows2_v[rb, pl.ds(g * LANES, LANES)])
                    a2 += [h, l]
                for j in range(1, L_SEQ):
                    for g in range(WG):
                        h, l = unpack(
                            rows1_v[rb + j, pl.ds(g * LANES, LANES)])
                        a1[2 * g] = a1[2 * g] + h
                        a1[2 * g + 1] = a1[2 * g + 1] + l
                        h, l = unpack(
                            rows2_v[rb + j, pl.ds(g * LANES, LANES)])
                        a2[2 * g] = a2[2 * g] + h
                        a2[2 * g + 1] = a2[2 * g + 1] + l
                dotv = a1[0] * a2[0]
                s1v = a1[0] * a1[0]
                s2v = a2[0] * a2[0]
                for g in range(1, 2 * WG):
                    dotv = dotv + a1[g] * a2[g]
                    s1v = s1v + a1[g] * a1[g]
                    s2v = s2v + a2[g] * a2[g]
                m = lane == r
                dot_t = jnp.where(m, jnp.sum(dotv), dot_t)
                s1_t = jnp.where(m, jnp.sum(s1v), s1_t)
                s2_t = jnp.where(m, jnp.sum(s2v), s2_t)
                return dot_t, s1_t, s2_t

            zeros = jnp.zeros((LANES,), jnp.float32)
            dot_t, s1_t, s2_t = lax.fori_loop(
                0, CB, row_body, (zeros, zeros, zeros))

            inv_l = 1.0 / float(L_SEQ)
            dot_m = dot_t * (inv_l * inv_l)
            s_m = (s1_t * s2_t) * (inv_l * inv_l * inv_l * inv_l)
            rs = _rsqrt_newton(s_m)
            sqrt_m = s_m * rs
            denom = jnp.maximum(sqrt_m, EPS)
            out_v[pl.ds(c * CB, CB)] = dot_m / denom

        start_chunk(0, 0)

        def pair_body(p, carry):
            c0 = 2 * p
            start_chunk(c0 + 1, 1)
            process_chunk(c0, 0)

            @pl.when(p + 1 < NCH // 2)
            def _():
                start_chunk(c0 + 2, 0)

            process_chunk(c0 + 1, 1)
            return carry

        lax.fori_loop(0, NCH // 2, pair_body, 0)
        pltpu.sync_copy(out_v, out_hbm.at[pl.ds(base, B_PER_W)])

    return _sc_cosine


@functools.cache
def _build_tc_relayout():
    grid = P_ROWS // VB + 1

    def pack16(x):
        t = x[:W, :].astype(jnp.bfloat16).astype(jnp.float32)
        b = x[W:, :].astype(jnp.bfloat16).astype(jnp.float32)
        ti = lax.bitcast_convert_type(t, jnp.int32)
        bi = lax.bitcast_convert_type(b, jnp.int32)
        return ti | lax.shift_right_logical(bi, 16)

    def body(x0_ref, x1_ref, x2_ref, x3_ref, o_ref):
        parts = [pack16(r[...]) for r in (x0_ref, x1_ref, x2_ref, x3_ref)]
        cat = jnp.concatenate(parts, axis=0)
        o_ref[...] = cat.T

    def spec(k):
        return pl.BlockSpec(
            (D, VB), lambda i, k=k: (0, jnp.minimum(4 * i + k, NBLK)))

    return pl.pallas_call(
        body,
        grid=(grid,),
        in_specs=[spec(0), spec(1), spec(2), spec(3)],
        out_specs=pl.BlockSpec((VB, 4 * W), lambda i: (i, 0)),
        out_shape=jax.ShapeDtypeStruct((P_ROWS, 4 * W), jnp.int32),
    )


def kernel(input1, input2, table):
    i1 = input1.reshape(-1)
    i2 = input2.reshape(-1)
    tabt = table.T
    packed = _build_tc_relayout()(tabt, tabt, tabt, tabt)
    tab2 = packed.reshape(N32, W)
    return _build_sc_cosine()(i1, i2, tab2)


# speedup vs baseline: 4.5772x; 1.1007x over previous
import functools

import jax
import jax.numpy as jnp
from jax import lax
from jax.experimental import pallas as pl
from jax.experimental.pallas import tpu as pltpu
from jax.experimental.pallas import tpu_sc as plsc

VOCAB = 1000000
D = 64
B = 16384
L_SEQ = 20
EPS = 1e-6

NC = 2
NS = 16
LANES = 16
NW = NC * NS
B_PER_W = B // NW
CB = 16
NCH = B_PER_W // CB
NI = CB * L_SEQ
W = D // 2
WG = W // LANES

VB = 4096
NBLK = VOCAB // VB
C_TAIL = NBLK * VB
P_ROWS = NBLK // 4 * VB + (VOCAB - C_TAIL)
N32 = 4 * P_ROWS


def _rsqrt_newton(x):
    i = plsc.bitcast(x, jnp.int32)
    i = jnp.int32(0x5F3759DF) - lax.shift_right_arithmetic(i, jnp.int32(1))
    y = plsc.bitcast(i, jnp.float32)
    for _ in range(3):
        y = y * (1.5 - 0.5 * (x * y) * y)
    return y


@functools.cache
def _build_sc_cosine():
    mesh = plsc.VectorSubcoreMesh(core_axis_name="c", subcore_axis_name="s")

    @functools.partial(
        pl.kernel,
        mesh=mesh,
        out_type=jax.ShapeDtypeStruct((B,), jnp.float32),
        compiler_params=pltpu.CompilerParams(
            needs_layout_passes=False, use_tc_tiling_on_sc=False),
        scratch_types=[
            pltpu.VMEM((L_SEQ, CB), jnp.int32),
            pltpu.VMEM((L_SEQ, CB), jnp.int32),
            pltpu.VMEM((NI,), jnp.int32),
            pltpu.VMEM((NI,), jnp.int32),
            pltpu.VMEM((NI, W), jnp.int32),
            pltpu.VMEM((NI, W), jnp.int32),
            pltpu.VMEM((L_SEQ, CB), jnp.int32),
            pltpu.VMEM((L_SEQ, CB), jnp.int32),
            pltpu.VMEM((NI,), jnp.int32),
            pltpu.VMEM((NI,), jnp.int32),
            pltpu.VMEM((NI, W), jnp.int32),
            pltpu.VMEM((NI, W), jnp.int32),
            pltpu.VMEM((B_PER_W,), jnp.float32),
            pltpu.SemaphoreType.DMA,
            pltpu.SemaphoreType.DMA,
            pltpu.SemaphoreType.DMA,
            pltpu.SemaphoreType.DMA,
        ],
    )
    def _sc_cosine(i1_hbm, i2_hbm, table_hbm, out_hbm,
                   idx1_a, idx2_a, idxp1_a, idxp2_a, rows1_a, rows2_a,
                   idx1_b, idx2_b, idxp1_b, idxp2_b, rows1_b, rows2_b,
                   out_v, sem1_a, sem2_a, sem1_b, sem2_b):
        wid = lax.axis_index("s") * NC + lax.axis_index("c")
        base = wid * B_PER_W
        hi_mask = jnp.int32(-65536)
        bufs = (
            (idx1_a, idx2_a, idxp1_a, idxp2_a, rows1_a, rows2_a,
             sem1_a, sem2_a),
            (idx1_b, idx2_b, idxp1_b, idxp2_b, rows1_b, rows2_b,
             sem1_b, sem2_b),
        )

        def unpack(wv):
            hi = plsc.bitcast(wv & hi_mask, jnp.float32)
            lo = plsc.bitcast(lax.shift_left(wv, 16), jnp.float32)
            return hi, lo

        def start_chunk(c, k):
            idx1_v, idx2_v, idxp1_v, idxp2_v, rows1_v, rows2_v, s1, s2 = \
                bufs[k]
            bcol = base + c * CB
            pltpu.sync_copy(i1_hbm.at[:, pl.ds(bcol, CB)], idx1_v)
            pltpu.sync_copy(i2_hbm.at[:, pl.ds(bcol, CB)], idx2_v)
            for t in range(L_SEQ):
                for src, dst in ((idx1_v, idxp1_v), (idx2_v, idxp2_v)):
                    iv = src[t, :]
                    r = iv & (4 * VB - 1)
                    main_lin = (iv - r) + 4 * (r & (VB - 1)) \
                        + lax.shift_right_logical(r, VB.bit_length() - 1)
                    tail_lin = iv * 4 - 3 * C_TAIL
                    dst[pl.ds(t * CB, CB)] = jnp.where(
                        iv < C_TAIL, main_lin, tail_lin)
            pltpu.async_copy(table_hbm.at[idxp1_v], rows1_v, s1)
            pltpu.async_copy(table_hbm.at[idxp2_v], rows2_v, s2)

        def process_chunk(c, k):
            _, _, idxp1_v, idxp2_v, rows1_v, rows2_v, s1, s2 = bufs[k]
            pltpu.make_async_copy(table_hbm.at[idxp1_v], rows1_v, s1).wait()
            pltpu.make_async_copy(table_hbm.at[idxp2_v], rows2_v, s2).wait()

            lane = lax.iota(jnp.int32, LANES)

            def row_body(r, carry2):
                dot_t, s1_t, s2_t = carry2
                a1 = []
                a2 = []
                for g in range(WG):
                    h, l = unpack(rows1_v[r, pl.ds(g * LANES, LANES)])
                    a1 += [h, l]
                    h, l = unpack(rows2_v[r, pl.ds(g * LANES, LANES)])
                    a2 += [h, l]
                for j in range(1, L_SEQ):
                    for g in range(WG):
                        h, l = unpack(
                            rows1_v[j * CB + r, pl.ds(g * LANES, LANES)])
                        a1[2 * g] = a1[2 * g] + h
                        a1[2 * g + 1] = a1[2 * g + 1] + l
                        h, l = unpack(
                            rows2_v[j * CB + r, pl.ds(g * LANES, LANES)])
                        a2[2 * g] = a2[2 * g] + h
                        a2[2 * g + 1] = a2[2 * g + 1] + l
                dotv = a1[0] * a2[0]
                s1v = a1[0] * a1[0]
                s2v = a2[0] * a2[0]
                for g in range(1, 2 * WG):
                    dotv = dotv + a1[g] * a2[g]
                    s1v = s1v + a1[g] * a1[g]
                    s2v = s2v + a2[g] * a2[g]
                m = lane == r
                dot_t = jnp.where(m, jnp.sum(dotv), dot_t)
                s1_t = jnp.where(m, jnp.sum(s1v), s1_t)
                s2_t = jnp.where(m, jnp.sum(s2v), s2_t)
                return dot_t, s1_t, s2_t

            zeros = jnp.zeros((LANES,), jnp.float32)
            dot_t, s1_t, s2_t = lax.fori_loop(
                0, CB, row_body, (zeros, zeros, zeros))

            inv_l = 1.0 / float(L_SEQ)
            dot_m = dot_t * (inv_l * inv_l)
            s_m = (s1_t * s2_t) * (inv_l * inv_l * inv_l * inv_l)
            rs = _rsqrt_newton(s_m)
            sqrt_m = s_m * rs
            denom = jnp.maximum(sqrt_m, EPS)
            out_v[pl.ds(c * CB, CB)] = dot_m / denom

        start_chunk(0, 0)

        def pair_body(p, carry):
            c0 = 2 * p
            start_chunk(c0 + 1, 1)
            process_chunk(c0, 0)

            @pl.when(p + 1 < NCH // 2)
            def _():
                start_chunk(c0 + 2, 0)

            process_chunk(c0 + 1, 1)
            return carry

        lax.fori_loop(0, NCH // 2, pair_body, 0)
        pltpu.sync_copy(out_v, out_hbm.at[pl.ds(base, B_PER_W)])

    return _sc_cosine


@functools.cache
def _build_tc_relayout():
    grid = P_ROWS // VB + 1

    def pack16(x):
        t = x[:W, :].astype(jnp.bfloat16).astype(jnp.float32)
        b = x[W:, :].astype(jnp.bfloat16).astype(jnp.float32)
        ti = lax.bitcast_convert_type(t, jnp.int32)
        bi = lax.bitcast_convert_type(b, jnp.int32)
        return ti | lax.shift_right_logical(bi, 16)

    def body(x0_ref, x1_ref, x2_ref, x3_ref, o_ref):
        parts = [pack16(r[...]) for r in (x0_ref, x1_ref, x2_ref, x3_ref)]
        cat = jnp.concatenate(parts, axis=0)
        o_ref[...] = cat.T

    def spec(k):
        return pl.BlockSpec(
            (D, VB), lambda i, k=k: (0, jnp.minimum(4 * i + k, NBLK)))

    return pl.pallas_call(
        body,
        grid=(grid,),
        in_specs=[spec(0), spec(1), spec(2), spec(3)],
        out_specs=pl.BlockSpec((VB, 4 * W), lambda i: (i, 0)),
        out_shape=jax.ShapeDtypeStruct((P_ROWS, 4 * W), jnp.int32),
    )


def kernel(input1, input2, table):
    tabt = table.T
    packed = _build_tc_relayout()(tabt, tabt, tabt, tabt)
    tab2 = packed.reshape(N32, W)
    return _build_sc_cosine()(input1.T, input2.T, tab2)
